# baseline (speedup 1.0000x reference)
import functools

import jax
import jax.numpy as jnp
from jax import lax
from jax.experimental import pallas as pl
from jax.experimental.pallas import tpu as pltpu
from jax.experimental.pallas import tpu_sc as plsc

D_MODEL = 256
D_Q = 16
NUM_DST = 6144
NUM_SRC = 384
N_TG = NUM_DST + NUM_SRC
E_BG = 196608
E_TG = 208896

NC = 2
NS = 16
NW = NC * NS

_HIGH = jax.lax.Precision.HIGHEST

_EPW = E_BG // NW
_EROWS = _EPW // 128


def _make_sc_bg_sums():
    mesh = plsc.VectorSubcoreMesh(core_axis_name="c", subcore_axis_name="s")

    @functools.partial(
        pl.kernel,
        out_type=(
            jax.ShapeDtypeStruct((NC, NUM_DST), jnp.float32),
            jax.ShapeDtypeStruct((NC, NUM_DST), jnp.float32),
        ),
        mesh=mesh,
        compiler_params=pltpu.CompilerParams(needs_layout_passes=False),
        scratch_types=[
            pltpu.VMEM((_EROWS, 128), jnp.int32),
            pltpu.VMEM((_EROWS, 128), jnp.float32),
            pltpu.VMEM((_EROWS, 128), jnp.float32),
            pltpu.VMEM((_EROWS, 128), jnp.float32),
            pltpu.VMEM((NUM_DST,), jnp.float32),
            pltpu.VMEM_SHARED((NUM_DST,), jnp.float32),
            pltpu.VMEM_SHARED((NUM_DST,), jnp.float32),
        ],
    )
    def k(dst_hbm, e_hbm, cnt_out, esum_out, dstv, ev, onesv, emv, zb,
          cnt_sh, esum_sh):
        c = lax.axis_index("c")
        s = lax.axis_index("s")
        w = c * NS + s
        row0 = w * _EROWS
        pltpu.sync_copy(dst_hbm.at[pl.ds(row0, _EROWS)], dstv)
        pltpu.sync_copy(e_hbm.at[pl.ds(row0, _EROWS)], ev)

        @pl.when(s == 0)
        def _():
            def zc(j, carry):
                zb[pl.ds(j * 16, 16)] = jnp.zeros((16,), jnp.float32)
                return carry
            lax.fori_loop(0, NUM_DST // 16, zc, 0)
            pltpu.sync_copy(zb, cnt_sh)
            pltpu.sync_copy(zb, esum_sh)

        def mk(j, carry):
            row = j // 8
            off = (j % 8) * 16
            e16 = ev[row, pl.ds(off, 16)]
            m = e16 > 0.0
            onesv[row, pl.ds(off, 16)] = jnp.where(m, 1.0, 0.0)
            emv[row, pl.ds(off, 16)] = jnp.where(m, e16, 0.0)
            return carry
        lax.fori_loop(0, _EPW // 16, mk, 0)

        plsc.subcore_barrier()

        def sc_row(i, carry):
            pltpu.sync_copy(onesv.at[i], cnt_sh.at[dstv.at[i]], add=True)
            pltpu.sync_copy(emv.at[i], esum_sh.at[dstv.at[i]], add=True)
            return carry
        lax.fori_loop(0, _EROWS, sc_row, 0)

        plsc.subcore_barrier()

        @pl.when(s == 0)
        def _():
            pltpu.sync_copy(cnt_sh, cnt_out.at[c])
            pltpu.sync_copy(esum_sh, esum_out.at[c])

    return k


def _make_sc_bg_pr():
    mesh = plsc.VectorSubcoreMesh(core_axis_name="c", subcore_axis_name="s")

    @functools.partial(
        pl.kernel,
        out_type=jax.ShapeDtypeStruct((NC, NUM_DST), jnp.float32),
        mesh=mesh,
        compiler_params=pltpu.CompilerParams(needs_layout_passes=False),
        scratch_types=[
            pltpu.VMEM((_EROWS, 128), jnp.int32),
            pltpu.VMEM((_EROWS, 128), jnp.int32),
            pltpu.VMEM((_EROWS, 128), jnp.float32),
            pltpu.VMEM((_EROWS, 128), jnp.float32),
            pltpu.VMEM((2 * NUM_DST // 128, 128), jnp.float32),
            pltpu.VMEM((2 * NUM_DST // 128, 128), jnp.float32),
            pltpu.VMEM((NUM_DST,), jnp.float32),
            pltpu.VMEM_SHARED((NUM_DST,), jnp.float32),
        ],
    )
    def k(src_hbm, dst_hbm, e_hbm, cntp_hbm, esump_hbm, pr_out,
          srcv, dstv, ev, valv, cnt2, esum2, divv, pr_sh):
        c = lax.axis_index("c")
        s = lax.axis_index("s")
        w = c * NS + s
        row0 = w * _EROWS
        pltpu.sync_copy(src_hbm.at[pl.ds(row0, _EROWS)], srcv)
        pltpu.sync_copy(dst_hbm.at[pl.ds(row0, _EROWS)], dstv)
        pltpu.sync_copy(e_hbm.at[pl.ds(row0, _EROWS)], ev)
        pltpu.sync_copy(cntp_hbm, cnt2)
        pltpu.sync_copy(esump_hbm, esum2)

        half = NUM_DST // 128

        def dv(j, carry):
            row = j // 8
            off = (j % 8) * 16
            cnt16 = cnt2[row, pl.ds(off, 16)] + cnt2[half + row, pl.ds(off, 16)]
            es16 = esum2[row, pl.ds(off, 16)] + esum2[half + row, pl.ds(off, 16)]
            div16 = jnp.where(cnt16 > 0.0, es16 / jnp.maximum(cnt16, 1.0), 0.0)
            divv[pl.ds(j * 16, 16)] = div16
            return carry
        lax.fori_loop(0, NUM_DST // 16, dv, 0)

        @pl.when(s == 0)
        def _():
            def zc(j, carry):
                pr_stage = jnp.zeros((16,), jnp.float32)
                valv[j // 8, pl.ds((j % 8) * 16, 16)] = pr_stage
                return carry
            lax.fori_loop(0, 128 // 16 * _EROWS, zc, 0)
            def zrow(i, carry):
                pltpu.sync_copy(valv.at[i], pr_sh.at[pl.ds(i * 128, 128)])
                return carry
            lax.fori_loop(0, _EROWS, zrow, 0)

        plsc.subcore_barrier()

        def ed(j, carry):
            row = j // 8
            off = (j % 8) * 16
            s16 = srcv[row, pl.ds(off, 16)]
            d16 = dstv[row, pl.ds(off, 16)]
            e16 = ev[row, pl.ds(off, 16)]
            m = e16 > 0.0
            dsrc = plsc.load_gather(divv, [s16])
            ddst = plsc.load_gather(divv, [d16])
            det = dsrc - ddst
            val = jnp.where(m & (det > 0.0), det, 0.0)
            valv[row, pl.ds(off, 16)] = val
            return carry
        lax.fori_loop(0, _EPW // 16, ed, 0)

        def sc_row(i, carry):
            pltpu.sync_copy(valv.at[i], pr_sh.at[dstv.at[i]], add=True)
            return carry
        lax.fori_loop(0, _EROWS, sc_row, 0)

        plsc.subcore_barrier()

        @pl.when(s == 0)
        def _():
            pltpu.sync_copy(pr_sh, pr_out.at[c])

    return k


_ETROWS = 104
_ETPT = _ETROWS * 128
_E_TG_PAD = NS * _ETPT
_BLK = 384
_NTU = 3328
_CELLS = _BLK * _NTU
_CPT = _CELLS // NS
_NPASS = NUM_DST // _BLK // NC
_CAP = 1024


def _make_sc_counts():
    mesh = plsc.VectorSubcoreMesh(core_axis_name="c", subcore_axis_name="s")

    @functools.partial(
        pl.kernel,
        out_type=jax.ShapeDtypeStruct((NUM_DST * _NTU,), jnp.int32),
        mesh=mesh,
        compiler_params=pltpu.CompilerParams(needs_layout_passes=False),
        scratch_types=[
            pltpu.VMEM((_ETROWS, 128), jnp.int32),
            pltpu.VMEM((_ETROWS, 128), jnp.int32),
            pltpu.VMEM((_ETROWS, 128), jnp.int32),
            pltpu.VMEM((_CAP // 128, 128), jnp.int32),
            pltpu.VMEM((_CAP // 128, 128), jnp.int32),
            pltpu.VMEM((_CAP // 128, 128), jnp.int32),
            pltpu.VMEM((_CAP // 128, 128), jnp.int32),
            pltpu.VMEM((_CAP,), jnp.int32),
            pltpu.VMEM_SHARED((_CELLS + 128,), jnp.int32),
            pltpu.SemaphoreType.DMA,
            pltpu.SemaphoreType.DMA,
        ],
    )
    def k(dst_hbm, src_hbm, z_hbm, c_out, dstv, srcv, valg, idxa, idxb,
          vala, valb, dumv, c_sh, sema, semb):
        c = lax.axis_index("c")
        s = lax.axis_index("s")
        pltpu.sync_copy(dst_hbm.at[pl.ds(s * _ETROWS, _ETROWS)], dstv)
        pltpu.sync_copy(src_hbm.at[pl.ds(s * _ETROWS, _ETROWS)], srcv)


        lane = lax.broadcasted_iota(jnp.int32, (16,), 0)
        _NGRP = _ETPT // _CAP

        def pre(j, carry):
            row = j // 8
            off = (j % 8) * 16
            s16 = srcv[row, pl.ds(off, 16)]
            d16 = dstv[row, pl.ds(off, 16)]
            hi = s16 >= (N_TG // 2)
            u16 = s16 - jnp.where(hi, N_TG // 2, 0)
            srcv[row, pl.ds(off, 16)] = ((d16 >> 3) * (_NTU * 8)
                                         + ((u16 >> 7) << 10)
                                         + ((d16 & 7) << 7)
                                         + (u16 & 127))
            valg[row, pl.ds(off, 16)] = jnp.where(hi, 65536, 1)
            return carry
        lax.fori_loop(0, _ETPT // 16, pre, 0)

        def do_pass(p, carry):
            blk = c * _NPASS + p
            lo = blk * _BLK
            pltpu.sync_copy(z_hbm.at[pl.ds(s * _CPT, _CPT)],
                            c_sh.at[pl.ds(s * _CPT, _CPT)])
            plsc.subcore_barrier()

            def fill(buf, vbuf, g):
                sh = lo * _NTU
                def rowloop(r, carry2):
                    grow = g * 8 + r
                    for qq in range(8):
                        d16 = dstv[grow, pl.ds(qq * 16, 16)]
                        tg = srcv[grow, pl.ds(qq * 16, 16)]
                        vg = valg[grow, pl.ds(qq * 16, 16)]
                        m = (d16 >= lo) & (d16 < lo + _BLK)
                        sacr = _CELLS + qq * 16 + lane
                        buf[r, pl.ds(qq * 16, 16)] = jnp.where(m, tg - sh,
                                                               sacr)
                        vbuf[r, pl.ds(qq * 16, 16)] = jnp.where(m, vg, 0)
                    return carry2
                lax.fori_loop(0, 8, rowloop, 0)

            def fire(buf, vbuf, sem):
                for i in range(_CAP // 128):
                    pltpu.async_copy(vbuf.at[i], c_sh.at[buf.at[i]], sem,
                                     add=True)

            def drain(sem):
                pltpu.make_async_copy(
                    z_hbm.at[pl.ds(0, _CAP)], dumv, sem).wait()

            fill(idxa, vala, 0)
            fire(idxa, vala, sema)
            fill(idxb, valb, 1)
            fire(idxb, valb, semb)

            def pair(t, carry2):
                drain(sema)
                fill(idxa, vala, 2 * t)
                fire(idxa, vala, sema)
                drain(semb)
                fill(idxb, valb, 2 * t + 1)
                fire(idxb, valb, semb)
                return carry2
            lax.fori_loop(1, (_NGRP - 1) // 2, pair, 0)

            drain(sema)
            fill(idxa, vala, _NGRP - 1)
            fire(idxa, vala, sema)
            drain(semb)
            drain(sema)

            plsc.subcore_barrier()
            pltpu.sync_copy(
                c_sh.at[pl.ds(s * _CPT, _CPT)],
                c_out.at[pl.ds(blk * _CELLS + s * _CPT, _CPT)])
            plsc.subcore_barrier()
            return carry

        lax.fori_loop(0, _NPASS, do_pass, 0)

    return k


_RCH = 512


def _pr_row(cntp, prp, j0, n):
    c0 = cntp[0:1, pl.ds(j0, n)]
    c1 = cntp[1:2, pl.ds(j0, n)]
    p0 = prp[0:1, pl.ds(j0, n)]
    p1 = prp[1:2, pl.ds(j0, n)]
    cnt = c0 + c1
    return jnp.where(cnt > 0.0, (p0 + p1) / jnp.maximum(cnt, 1.0), 0.0)


def _topk_body(cntp_ref, prp_ref, cntpt_ref, prpt_ref, xe_ref, wdis_ref,
               src_out, qall_out):
    prr = _pr_row(cntp_ref, prp_ref, 0, NUM_DST)

    def rk(i, acc):
        i0 = i * _RCH
        c0 = cntpt_ref[pl.ds(i0, _RCH), 0:1]
        c1 = cntpt_ref[pl.ds(i0, _RCH), 1:2]
        p0 = prpt_ref[pl.ds(i0, _RCH), 0:1]
        p1 = prpt_ref[pl.ds(i0, _RCH), 1:2]
        cnt = c0 + c1
        pri = jnp.where(cnt > 0.0, (p0 + p1) / jnp.maximum(cnt, 1.0), 0.0)
        iidx = lax.broadcasted_iota(jnp.int32, (_RCH, 1), 0) + i0
        jidx = lax.broadcasted_iota(jnp.int32, (1, NUM_DST), 1)
        lt = pri < prr
        eqlow = (pri == prr) & (iidx < jidx)
        contrib = jnp.where(lt | eqlow, 1.0, 0.0)
        return acc + jnp.sum(contrib, axis=0, keepdims=True)

    rank = lax.fori_loop(0, NUM_DST // _RCH, rk,
                         jnp.zeros((1, NUM_DST), jnp.float32))
    riota = lax.broadcasted_iota(jnp.int32, (NUM_SRC, NUM_DST), 0)
    psel = jnp.where(rank.astype(jnp.int32) == riota, 1.0, 0.0)
    xe = xe_ref[...]
    src = lax.dot_general(psel, xe, (((1,), (0,)), ((), ())),
                          precision=_HIGH, preferred_element_type=jnp.float32)
    src_out[...] = src
    wdis = wdis_ref[...]
    q_dst = lax.dot_general(xe, wdis, (((1,), (0,)), ((), ())),
                            precision=_HIGH,
                            preferred_element_type=jnp.float32) * 0.25
    q_src = lax.dot_general(src, wdis, (((1,), (0,)), ((), ())),
                            precision=_HIGH,
                            preferred_element_type=jnp.float32) * 0.25
    qall_out[...] = jnp.concatenate([q_dst, q_src], axis=0)


def _make_topk(interpret=False):
    return pl.pallas_call(
        _topk_body,
        out_shape=(
            jax.ShapeDtypeStruct((NUM_SRC, D_MODEL), jnp.float32),
            jax.ShapeDtypeStruct((N_TG, D_Q), jnp.float32),
        ),
        interpret=interpret,
    )


_NUT = _NTU // 128
_NDG = _BLK // 8
_CROWS = _CELLS // 128
_NHALF = N_TG // 2


def _attn_body(qd_ref, qe_ref, qo_ref, c_ref, xee_ref, xeo_ref, xed_ref,
               out_ref):
    ge = lax.dot_general(qd_ref[...], qe_ref[...], (((1,), (1,)), ((), ())),
                         precision=_HIGH, preferred_element_type=jnp.float32)
    go = lax.dot_general(qd_ref[...], qo_ref[...], (((1,), (1,)), ((), ())),
                         precision=_HIGH, preferred_element_type=jnp.float32)
    c4 = c_ref[...].reshape(_NDG, _NUT, 8, 128)
    pe_pieces = []
    po_pieces = []
    a_sum = jnp.zeros((_BLK, 1), jnp.float32)
    r_sum = jnp.zeros((_BLK, 1), jnp.float32)
    for ut in range(_NUT):
        c32 = c4[:, ut, :, :].reshape(_BLK, 128)
        ce = (c32 & 0xFFFF).astype(jnp.float32)
        co = (c32 >> 16).astype(jnp.float32)
        geu = ge[:, ut * 128:(ut + 1) * 128]
        gou = go[:, ut * 128:(ut + 1) * 128]
        pst = ce * jnp.maximum(geu, 0.0)
        pso = co * jnp.maximum(gou, 0.0)
        a_sum = a_sum + (jnp.sum(pst, axis=1, keepdims=True)
                         + jnp.sum(pso, axis=1, keepdims=True))
        r_sum = r_sum + (jnp.sum(ce * geu, axis=1, keepdims=True)
                         + jnp.sum(co * gou, axis=1, keepdims=True))
        pe_pieces.append(pst)
        po_pieces.append(pso)
    r_sum = a_sum - r_sum
    pe = jnp.concatenate(pe_pieces, axis=1)
    po = jnp.concatenate(po_pieces, axis=1)
    xea = (lax.dot_general(pe, xee_ref[...], (((1,), (0,)), ((), ())),
                           precision=jax.lax.Precision.DEFAULT,
                           preferred_element_type=jnp.float32)
           + lax.dot_general(po, xeo_ref[...], (((1,), (0,)), ((), ())),
                             precision=jax.lax.Precision.DEFAULT,
                             preferred_element_type=jnp.float32))
    denom = a_sum + r_sum + 0.01
    out_ref[...] = (xea + xed_ref[...] * r_sum) / denom


def _make_attn(interpret=False):
    return pl.pallas_call(
        _attn_body,
        grid=(NUM_DST // _BLK,),
        in_specs=[
            pl.BlockSpec((_BLK, D_Q), lambda i: (i, 0)),
            pl.BlockSpec((_NTU, D_Q), lambda i: (0, 0)),
            pl.BlockSpec((_NTU, D_Q), lambda i: (0, 0)),
            pl.BlockSpec((_CROWS, 128), lambda i: (i, 0)),
            pl.BlockSpec((_NTU, D_MODEL), lambda i: (0, 0)),
            pl.BlockSpec((_NTU, D_MODEL), lambda i: (0, 0)),
            pl.BlockSpec((_BLK, D_MODEL), lambda i: (i, 0)),
        ],
        out_specs=pl.BlockSpec((_BLK, D_MODEL), lambda i: (i, 0)),
        out_shape=jax.ShapeDtypeStruct((NUM_DST, D_MODEL), jnp.float32),
        interpret=interpret,
    )


def _bn_in(x, g, b, eps=1e-5):
    m = jnp.mean(x, axis=0, keepdims=True)
    v = jnp.mean((x - m) * (x - m), axis=0, keepdims=True)
    return (x - m) / jnp.sqrt(v + eps) * g + b


def _mlp_body(xe_ref, xt_ref, bng_ref, bnb_ref, w1_ref, b1_ref, g1_ref,
              bt1_ref, w2_ref, b2_ref, g2_ref, bt2_ref, out_ref):
    xt = xt_ref[...]
    x = xe_ref[...] + _bn_in(xt, bng_ref[...], bnb_ref[...])
    y1 = lax.dot_general(x, w1_ref[...], (((1,), (0,)), ((), ())),
                         precision=_HIGH,
                         preferred_element_type=jnp.float32) + b1_ref[...]
    h = jnp.maximum(_bn_in(y1, g1_ref[...], bt1_ref[...]), 0.0)
    y2 = lax.dot_general(h, w2_ref[...], (((1,), (0,)), ((), ())),
                         precision=_HIGH,
                         preferred_element_type=jnp.float32) + b2_ref[...]
    h2 = _bn_in(y2, g2_ref[...], bt2_ref[...])
    out_ref[...] = x + h2


def _make_mlp(interpret=False):
    return pl.pallas_call(
        _mlp_body,
        out_shape=jax.ShapeDtypeStruct((NUM_DST, D_MODEL), jnp.float32),
        interpret=interpret,
    )


def kernel(Xe, bg_edge_index, bg_E, Tg_edge_index, shape, W_dis, bn_g, bn_b,
           W1, b1, g1, bt1, W2, b2, g2, bt2):
    src_b2 = bg_edge_index[0].reshape(E_BG // 128, 128)
    dst_b2 = bg_edge_index[1].reshape(E_BG // 128, 128)
    e2 = bg_E.reshape(E_BG // 128, 128)

    cnt_p, esum_p = _make_sc_bg_sums()(dst_b2, e2)
    pr_p = _make_sc_bg_pr()(src_b2, dst_b2, e2,
                            cnt_p.reshape(2 * NUM_DST // 128, 128),
                            esum_p.reshape(2 * NUM_DST // 128, 128))

    pad = _E_TG_PAD - E_TG
    src_t2 = jnp.concatenate(
        [Tg_edge_index[0], jnp.zeros((pad,), jnp.int32)]).reshape(
            _E_TG_PAD // 128, 128)
    dst_t2 = jnp.concatenate(
        [Tg_edge_index[1], jnp.full((pad,), -1, jnp.int32)]).reshape(
            _E_TG_PAD // 128, 128)
    zeros_hbm = jnp.zeros((_CELLS,), jnp.int32) + (pr_p[0, 0] * 0.0).astype(jnp.int32)
    c_flat = _make_sc_counts()(dst_t2, src_t2, zeros_hbm)
    C = c_flat.reshape(NUM_DST * _NTU // 128, 128)

    cnt_pt = cnt_p.T
    pr_pt = pr_p.T
    SRC, q_all = _make_topk()(cnt_p, pr_p, cnt_pt, pr_pt, Xe, W_dis)

    xe_all = jnp.concatenate([Xe, SRC], axis=0)
    q_dst = q_all[:NUM_DST]
    padu = _NTU - N_TG // 2
    half = N_TG // 2
    q_e = jnp.concatenate([q_all[:half], jnp.zeros((padu, D_Q), jnp.float32)])
    q_o = jnp.concatenate([q_all[half:], jnp.zeros((padu, D_Q), jnp.float32)])
    xe_e = jnp.concatenate(
        [xe_all[:half], jnp.zeros((padu, D_MODEL), jnp.float32)])
    xe_o = jnp.concatenate(
        [xe_all[half:], jnp.zeros((padu, D_MODEL), jnp.float32)])
    xe_trans = _make_attn()(q_dst, q_e, q_o, C, xe_e, xe_o, Xe)

    out = _make_mlp()(Xe, xe_trans,
                      bn_g.reshape(1, D_MODEL), bn_b.reshape(1, D_MODEL),
                      W1, b1.reshape(1, D_MODEL), g1.reshape(1, D_MODEL),
                      bt1.reshape(1, D_MODEL),
                      W2, b2.reshape(1, D_MODEL), g2.reshape(1, D_MODEL),
                      bt2.reshape(1, D_MODEL))
    return out


# speedup vs baseline: 1.0799x; 1.0799x over previous
import functools

import jax
import jax.numpy as jnp
from jax import lax
from jax.experimental import pallas as pl
from jax.experimental.pallas import tpu as pltpu
from jax.experimental.pallas import tpu_sc as plsc

D_MODEL = 256
D_Q = 16
NUM_DST = 6144
NUM_SRC = 384
N_TG = NUM_DST + NUM_SRC
E_BG = 196608
E_TG = 208896

NC = 2
NS = 16
NW = NC * NS

_HIGH = jax.lax.Precision.HIGHEST

_EPW = E_BG // NW
_EROWS = _EPW // 128


def _make_sc_bg_sums():
    mesh = plsc.VectorSubcoreMesh(core_axis_name="c", subcore_axis_name="s")

    @functools.partial(
        pl.kernel,
        out_type=(
            jax.ShapeDtypeStruct((NC, NUM_DST), jnp.float32),
            jax.ShapeDtypeStruct((NC, NUM_DST), jnp.float32),
        ),
        mesh=mesh,
        compiler_params=pltpu.CompilerParams(needs_layout_passes=False),
        scratch_types=[
            pltpu.VMEM((_EROWS, 128), jnp.int32),
            pltpu.VMEM((_EROWS, 128), jnp.float32),
            pltpu.VMEM((_EROWS, 128), jnp.float32),
            pltpu.VMEM((_EROWS, 128), jnp.float32),
            pltpu.VMEM((NUM_DST,), jnp.float32),
            pltpu.VMEM_SHARED((NUM_DST,), jnp.float32),
            pltpu.VMEM_SHARED((NUM_DST,), jnp.float32),
        ],
    )
    def k(dst_hbm, e_hbm, cnt_out, esum_out, dstv, ev, onesv, emv, zb,
          cnt_sh, esum_sh):
        c = lax.axis_index("c")
        s = lax.axis_index("s")
        w = c * NS + s
        row0 = w * _EROWS
        pltpu.sync_copy(dst_hbm.at[pl.ds(row0, _EROWS)], dstv)
        pltpu.sync_copy(e_hbm.at[pl.ds(row0, _EROWS)], ev)

        @pl.when(s == 0)
        def _():
            def zc(j, carry):
                zb[pl.ds(j * 16, 16)] = jnp.zeros((16,), jnp.float32)
                return carry
            lax.fori_loop(0, NUM_DST // 16, zc, 0)
            pltpu.sync_copy(zb, cnt_sh)
            pltpu.sync_copy(zb, esum_sh)

        def mk(j, carry):
            row = j // 8
            off = (j % 8) * 16
            e16 = ev[row, pl.ds(off, 16)]
            m = e16 > 0.0
            onesv[row, pl.ds(off, 16)] = jnp.where(m, 1.0, 0.0)
            emv[row, pl.ds(off, 16)] = jnp.where(m, e16, 0.0)
            return carry
        lax.fori_loop(0, _EPW // 16, mk, 0)

        plsc.subcore_barrier()

        def sc_row(i, carry):
            pltpu.sync_copy(onesv.at[i], cnt_sh.at[dstv.at[i]], add=True)
            pltpu.sync_copy(emv.at[i], esum_sh.at[dstv.at[i]], add=True)
            return carry
        lax.fori_loop(0, _EROWS, sc_row, 0)

        plsc.subcore_barrier()

        @pl.when(s == 0)
        def _():
            pltpu.sync_copy(cnt_sh, cnt_out.at[c])
            pltpu.sync_copy(esum_sh, esum_out.at[c])

    return k


def _make_sc_bg_pr():
    mesh = plsc.VectorSubcoreMesh(core_axis_name="c", subcore_axis_name="s")

    @functools.partial(
        pl.kernel,
        out_type=jax.ShapeDtypeStruct((NC, NUM_DST), jnp.float32),
        mesh=mesh,
        compiler_params=pltpu.CompilerParams(needs_layout_passes=False),
        scratch_types=[
            pltpu.VMEM((_EROWS, 128), jnp.int32),
            pltpu.VMEM((_EROWS, 128), jnp.int32),
            pltpu.VMEM((_EROWS, 128), jnp.float32),
            pltpu.VMEM((_EROWS, 128), jnp.float32),
            pltpu.VMEM((2 * NUM_DST // 128, 128), jnp.float32),
            pltpu.VMEM((2 * NUM_DST // 128, 128), jnp.float32),
            pltpu.VMEM((NUM_DST,), jnp.float32),
            pltpu.VMEM_SHARED((NUM_DST,), jnp.float32),
        ],
    )
    def k(src_hbm, dst_hbm, e_hbm, cntp_hbm, esump_hbm, pr_out,
          srcv, dstv, ev, valv, cnt2, esum2, divv, pr_sh):
        c = lax.axis_index("c")
        s = lax.axis_index("s")
        w = c * NS + s
        row0 = w * _EROWS
        pltpu.sync_copy(src_hbm.at[pl.ds(row0, _EROWS)], srcv)
        pltpu.sync_copy(dst_hbm.at[pl.ds(row0, _EROWS)], dstv)
        pltpu.sync_copy(e_hbm.at[pl.ds(row0, _EROWS)], ev)
        pltpu.sync_copy(cntp_hbm, cnt2)
        pltpu.sync_copy(esump_hbm, esum2)

        half = NUM_DST // 128

        def dv(j, carry):
            row = j // 8
            off = (j % 8) * 16
            cnt16 = cnt2[row, pl.ds(off, 16)] + cnt2[half + row, pl.ds(off, 16)]
            es16 = esum2[row, pl.ds(off, 16)] + esum2[half + row, pl.ds(off, 16)]
            div16 = jnp.where(cnt16 > 0.0, es16 / jnp.maximum(cnt16, 1.0), 0.0)
            divv[pl.ds(j * 16, 16)] = div16
            return carry
        lax.fori_loop(0, NUM_DST // 16, dv, 0)

        @pl.when(s == 0)
        def _():
            def zc(j, carry):
                pr_stage = jnp.zeros((16,), jnp.float32)
                valv[j // 8, pl.ds((j % 8) * 16, 16)] = pr_stage
                return carry
            lax.fori_loop(0, 128 // 16 * _EROWS, zc, 0)
            def zrow(i, carry):
                pltpu.sync_copy(valv.at[i], pr_sh.at[pl.ds(i * 128, 128)])
                return carry
            lax.fori_loop(0, _EROWS, zrow, 0)

        plsc.subcore_barrier()

        def ed(j, carry):
            row = j // 8
            off = (j % 8) * 16
            s16 = srcv[row, pl.ds(off, 16)]
            d16 = dstv[row, pl.ds(off, 16)]
            e16 = ev[row, pl.ds(off, 16)]
            m = e16 > 0.0
            dsrc = plsc.load_gather(divv, [s16])
            ddst = plsc.load_gather(divv, [d16])
            det = dsrc - ddst
            val = jnp.where(m & (det > 0.0), det, 0.0)
            valv[row, pl.ds(off, 16)] = val
            return carry
        lax.fori_loop(0, _EPW // 16, ed, 0)

        def sc_row(i, carry):
            pltpu.sync_copy(valv.at[i], pr_sh.at[dstv.at[i]], add=True)
            return carry
        lax.fori_loop(0, _EROWS, sc_row, 0)

        plsc.subcore_barrier()

        @pl.when(s == 0)
        def _():
            pltpu.sync_copy(pr_sh, pr_out.at[c])

    return k


_ETROWS = 104
_ETPT = _ETROWS * 128
_E_TG_PAD = NS * _ETPT
_BLK = 384
_NTU = 3328
_CELLS = _BLK * _NTU
_CPT = _CELLS // NS
_NBLK_H = NUM_DST // _BLK // 2
_NPASS = _NBLK_H // NC
_CAP = 1024


def _make_sc_counts(half):
    mesh = plsc.VectorSubcoreMesh(core_axis_name="c", subcore_axis_name="s")

    @functools.partial(
        pl.kernel,
        out_type=jax.ShapeDtypeStruct((NUM_DST * _NTU // 2,), jnp.int32),
        mesh=mesh,
        compiler_params=pltpu.CompilerParams(needs_layout_passes=False),
        scratch_types=[
            pltpu.VMEM((_ETROWS, 128), jnp.int32),
            pltpu.VMEM((_ETROWS, 128), jnp.int32),
            pltpu.VMEM((_ETROWS, 128), jnp.int32),
            pltpu.VMEM((_CAP // 128, 128), jnp.int32),
            pltpu.VMEM((_CAP // 128, 128), jnp.int32),
            pltpu.VMEM((_CAP // 128, 128), jnp.int32),
            pltpu.VMEM((_CAP // 128, 128), jnp.int32),
            pltpu.VMEM((_CAP,), jnp.int32),
            pltpu.VMEM_SHARED((_CELLS + 128,), jnp.int32),
            pltpu.SemaphoreType.DMA,
            pltpu.SemaphoreType.DMA,
        ],
    )
    def k(dst_hbm, src_hbm, z_hbm, c_out, dstv, srcv, valg, idxa, idxb,
          vala, valb, dumv, c_sh, sema, semb):
        c = lax.axis_index("c")
        s = lax.axis_index("s")
        pltpu.sync_copy(dst_hbm.at[pl.ds(s * _ETROWS, _ETROWS)], dstv)
        pltpu.sync_copy(src_hbm.at[pl.ds(s * _ETROWS, _ETROWS)], srcv)


        lane = lax.broadcasted_iota(jnp.int32, (16,), 0)
        _NGRP = _ETPT // _CAP

        def pre(j, carry):
            row = j // 8
            off = (j % 8) * 16
            s16 = srcv[row, pl.ds(off, 16)]
            d16 = dstv[row, pl.ds(off, 16)]
            hi = s16 >= (N_TG // 2)
            u16 = s16 - jnp.where(hi, N_TG // 2, 0)
            srcv[row, pl.ds(off, 16)] = ((d16 >> 3) * (_NTU * 8)
                                         + ((u16 >> 7) << 10)
                                         + ((d16 & 7) << 7)
                                         + (u16 & 127))
            valg[row, pl.ds(off, 16)] = jnp.where(hi, 65536, 1)
            return carry
        lax.fori_loop(0, _ETPT // 16, pre, 0)

        def do_pass(p, carry):
            blk = c * _NPASS + p
            lo = (half * _NBLK_H + blk) * _BLK
            pltpu.sync_copy(z_hbm.at[pl.ds(s * _CPT, _CPT)],
                            c_sh.at[pl.ds(s * _CPT, _CPT)])
            plsc.subcore_barrier()

            def fill(buf, vbuf, g):
                sh = lo * _NTU
                def rowloop(r, carry2):
                    grow = g * 8 + r
                    for qq in range(8):
                        d16 = dstv[grow, pl.ds(qq * 16, 16)]
                        tg = srcv[grow, pl.ds(qq * 16, 16)]
                        vg = valg[grow, pl.ds(qq * 16, 16)]
                        m = (d16 >= lo) & (d16 < lo + _BLK)
                        sacr = _CELLS + qq * 16 + lane
                        buf[r, pl.ds(qq * 16, 16)] = jnp.where(m, tg - sh,
                                                               sacr)
                        vbuf[r, pl.ds(qq * 16, 16)] = jnp.where(m, vg, 0)
                    return carry2
                lax.fori_loop(0, 8, rowloop, 0)

            def fire(buf, vbuf, sem):
                for i in range(_CAP // 128):
                    pltpu.async_copy(vbuf.at[i], c_sh.at[buf.at[i]], sem,
                                     add=True)

            def drain(sem):
                pltpu.make_async_copy(
                    z_hbm.at[pl.ds(0, _CAP)], dumv, sem).wait()

            fill(idxa, vala, 0)
            fire(idxa, vala, sema)
            fill(idxb, valb, 1)
            fire(idxb, valb, semb)

            def pair(t, carry2):
                drain(sema)
                fill(idxa, vala, 2 * t)
                fire(idxa, vala, sema)
                drain(semb)
                fill(idxb, valb, 2 * t + 1)
                fire(idxb, valb, semb)
                return carry2
            lax.fori_loop(1, (_NGRP - 1) // 2, pair, 0)

            drain(sema)
            fill(idxa, vala, _NGRP - 1)
            fire(idxa, vala, sema)
            drain(semb)
            drain(sema)

            plsc.subcore_barrier()
            pltpu.sync_copy(
                c_sh.at[pl.ds(s * _CPT, _CPT)],
                c_out.at[pl.ds(blk * _CELLS + s * _CPT, _CPT)])
            plsc.subcore_barrier()
            return carry

        lax.fori_loop(0, _NPASS, do_pass, 0)

    return k


_RCH = 512


def _pr_row(cntp, prp, j0, n):
    c0 = cntp[0:1, pl.ds(j0, n)]
    c1 = cntp[1:2, pl.ds(j0, n)]
    p0 = prp[0:1, pl.ds(j0, n)]
    p1 = prp[1:2, pl.ds(j0, n)]
    cnt = c0 + c1
    return jnp.where(cnt > 0.0, (p0 + p1) / jnp.maximum(cnt, 1.0), 0.0)


def _topk_body(cntp_ref, prp_ref, cntpt_ref, prpt_ref, xe_ref, wdis_ref,
               src_out, qall_out):
    prr = _pr_row(cntp_ref, prp_ref, 0, NUM_DST)

    def rk(i, acc):
        i0 = i * _RCH
        c0 = cntpt_ref[pl.ds(i0, _RCH), 0:1]
        c1 = cntpt_ref[pl.ds(i0, _RCH), 1:2]
        p0 = prpt_ref[pl.ds(i0, _RCH), 0:1]
        p1 = prpt_ref[pl.ds(i0, _RCH), 1:2]
        cnt = c0 + c1
        pri = jnp.where(cnt > 0.0, (p0 + p1) / jnp.maximum(cnt, 1.0), 0.0)
        iidx = lax.broadcasted_iota(jnp.int32, (_RCH, 1), 0) + i0
        jidx = lax.broadcasted_iota(jnp.int32, (1, NUM_DST), 1)
        lt = pri < prr
        eqlow = (pri == prr) & (iidx < jidx)
        contrib = jnp.where(lt | eqlow, 1.0, 0.0)
        return acc + jnp.sum(contrib, axis=0, keepdims=True)

    rank = lax.fori_loop(0, NUM_DST // _RCH, rk,
                         jnp.zeros((1, NUM_DST), jnp.float32))
    riota = lax.broadcasted_iota(jnp.int32, (NUM_SRC, NUM_DST), 0)
    psel = jnp.where(rank.astype(jnp.int32) == riota, 1.0, 0.0)
    xe = xe_ref[...]
    src = lax.dot_general(psel, xe, (((1,), (0,)), ((), ())),
                          precision=_HIGH, preferred_element_type=jnp.float32)
    src_out[...] = src
    wdis = wdis_ref[...]
    q_dst = lax.dot_general(xe, wdis, (((1,), (0,)), ((), ())),
                            precision=_HIGH,
                            preferred_element_type=jnp.float32) * 0.25
    q_src = lax.dot_general(src, wdis, (((1,), (0,)), ((), ())),
                            precision=_HIGH,
                            preferred_element_type=jnp.float32) * 0.25
    qall_out[...] = jnp.concatenate([q_dst, q_src], axis=0)


def _make_topk(interpret=False):
    return pl.pallas_call(
        _topk_body,
        out_shape=(
            jax.ShapeDtypeStruct((NUM_SRC, D_MODEL), jnp.float32),
            jax.ShapeDtypeStruct((N_TG, D_Q), jnp.float32),
        ),
        interpret=interpret,
    )


_NUT = _NTU // 128
_NDG = _BLK // 8
_CROWS = _CELLS // 128
_NHALF = N_TG // 2


def _attn_body(qd_ref, qe_ref, qo_ref, c_ref, xee_ref, xeo_ref, xed_ref,
               out_ref):
    ge = lax.dot_general(qd_ref[...], qe_ref[...], (((1,), (1,)), ((), ())),
                         precision=_HIGH, preferred_element_type=jnp.float32)
    go = lax.dot_general(qd_ref[...], qo_ref[...], (((1,), (1,)), ((), ())),
                         precision=_HIGH, preferred_element_type=jnp.float32)
    c4 = c_ref[...].reshape(_NDG, _NUT, 8, 128)
    pe_pieces = []
    po_pieces = []
    a_sum = jnp.zeros((_BLK, 1), jnp.float32)
    r_sum = jnp.zeros((_BLK, 1), jnp.float32)
    for ut in range(_NUT):
        c32 = c4[:, ut, :, :].reshape(_BLK, 128)
        ce = (c32 & 0xFFFF).astype(jnp.float32)
        co = (c32 >> 16).astype(jnp.float32)
        geu = ge[:, ut * 128:(ut + 1) * 128]
        gou = go[:, ut * 128:(ut + 1) * 128]
        pst = ce * jnp.maximum(geu, 0.0)
        pso = co * jnp.maximum(gou, 0.0)
        a_sum = a_sum + (jnp.sum(pst, axis=1, keepdims=True)
                         + jnp.sum(pso, axis=1, keepdims=True))
        r_sum = r_sum + (jnp.sum(ce * geu, axis=1, keepdims=True)
                         + jnp.sum(co * gou, axis=1, keepdims=True))
        pe_pieces.append(pst)
        po_pieces.append(pso)
    r_sum = a_sum - r_sum
    pe = jnp.concatenate(pe_pieces, axis=1)
    po = jnp.concatenate(po_pieces, axis=1)
    xea = (lax.dot_general(pe, xee_ref[...], (((1,), (0,)), ((), ())),
                           precision=jax.lax.Precision.DEFAULT,
                           preferred_element_type=jnp.float32)
           + lax.dot_general(po, xeo_ref[...], (((1,), (0,)), ((), ())),
                             precision=jax.lax.Precision.DEFAULT,
                             preferred_element_type=jnp.float32))
    denom = a_sum + r_sum + 0.01
    out_ref[...] = (xea + xed_ref[...] * r_sum) / denom


def _make_attn(interpret=False):
    return pl.pallas_call(
        _attn_body,
        grid=(NUM_DST // _BLK // 2,),
        in_specs=[
            pl.BlockSpec((_BLK, D_Q), lambda i: (i, 0)),
            pl.BlockSpec((_NTU, D_Q), lambda i: (0, 0)),
            pl.BlockSpec((_NTU, D_Q), lambda i: (0, 0)),
            pl.BlockSpec((_CROWS, 128), lambda i: (i, 0)),
            pl.BlockSpec((_NTU, D_MODEL), lambda i: (0, 0)),
            pl.BlockSpec((_NTU, D_MODEL), lambda i: (0, 0)),
            pl.BlockSpec((_BLK, D_MODEL), lambda i: (i, 0)),
        ],
        out_specs=pl.BlockSpec((_BLK, D_MODEL), lambda i: (i, 0)),
        out_shape=jax.ShapeDtypeStruct((NUM_DST // 2, D_MODEL), jnp.float32),
        interpret=interpret,
    )


def _bn_in(x, g, b, eps=1e-5):
    m = jnp.mean(x, axis=0, keepdims=True)
    v = jnp.mean((x - m) * (x - m), axis=0, keepdims=True)
    return (x - m) / jnp.sqrt(v + eps) * g + b


def _mlp_body(xe_ref, xt_ref, bng_ref, bnb_ref, w1_ref, b1_ref, g1_ref,
              bt1_ref, w2_ref, b2_ref, g2_ref, bt2_ref, out_ref):
    xt = xt_ref[...]
    x = xe_ref[...] + _bn_in(xt, bng_ref[...], bnb_ref[...])
    y1 = lax.dot_general(x, w1_ref[...], (((1,), (0,)), ((), ())),
                         precision=_HIGH,
                         preferred_element_type=jnp.float32) + b1_ref[...]
    h = jnp.maximum(_bn_in(y1, g1_ref[...], bt1_ref[...]), 0.0)
    y2 = lax.dot_general(h, w2_ref[...], (((1,), (0,)), ((), ())),
                         precision=_HIGH,
                         preferred_element_type=jnp.float32) + b2_ref[...]
    h2 = _bn_in(y2, g2_ref[...], bt2_ref[...])
    out_ref[...] = x + h2


def _make_mlp(interpret=False):
    return pl.pallas_call(
        _mlp_body,
        out_shape=jax.ShapeDtypeStruct((NUM_DST, D_MODEL), jnp.float32),
        interpret=interpret,
    )


def kernel(Xe, bg_edge_index, bg_E, Tg_edge_index, shape, W_dis, bn_g, bn_b,
           W1, b1, g1, bt1, W2, b2, g2, bt2):
    src_b2 = bg_edge_index[0].reshape(E_BG // 128, 128)
    dst_b2 = bg_edge_index[1].reshape(E_BG // 128, 128)
    e2 = bg_E.reshape(E_BG // 128, 128)

    cnt_p, esum_p = _make_sc_bg_sums()(dst_b2, e2)
    pr_p = _make_sc_bg_pr()(src_b2, dst_b2, e2,
                            cnt_p.reshape(2 * NUM_DST // 128, 128),
                            esum_p.reshape(2 * NUM_DST // 128, 128))

    pad = _E_TG_PAD - E_TG
    src_t2 = jnp.concatenate(
        [Tg_edge_index[0], jnp.zeros((pad,), jnp.int32)]).reshape(
            _E_TG_PAD // 128, 128)
    dst_t2 = jnp.concatenate(
        [Tg_edge_index[1], jnp.full((pad,), -1, jnp.int32)]).reshape(
            _E_TG_PAD // 128, 128)
    zeros_hbm = jnp.zeros((_CELLS,), jnp.int32) + (pr_p[0, 0] * 0.0).astype(jnp.int32)
    c_lo = _make_sc_counts(0)(dst_t2, src_t2, zeros_hbm)
    c_hi = _make_sc_counts(1)(dst_t2, src_t2, zeros_hbm)
    C_lo = c_lo.reshape(NUM_DST * _NTU // 256, 128)
    C_hi = c_hi.reshape(NUM_DST * _NTU // 256, 128)

    cnt_pt = cnt_p.T
    pr_pt = pr_p.T
    SRC, q_all = _make_topk()(cnt_p, pr_p, cnt_pt, pr_pt, Xe, W_dis)

    xe_all = jnp.concatenate([Xe, SRC], axis=0)
    q_dst = q_all[:NUM_DST]
    padu = _NTU - N_TG // 2
    half = N_TG // 2
    q_e = jnp.concatenate([q_all[:half], jnp.zeros((padu, D_Q), jnp.float32)])
    q_o = jnp.concatenate([q_all[half:], jnp.zeros((padu, D_Q), jnp.float32)])
    xe_e = jnp.concatenate(
        [xe_all[:half], jnp.zeros((padu, D_MODEL), jnp.float32)])
    xe_o = jnp.concatenate(
        [xe_all[half:], jnp.zeros((padu, D_MODEL), jnp.float32)])
    hd = NUM_DST // 2
    attn = _make_attn()
    xt_lo = attn(q_dst[:hd], q_e, q_o, C_lo, xe_e, xe_o, Xe[:hd])
    xt_hi = attn(q_dst[hd:], q_e, q_o, C_hi, xe_e, xe_o, Xe[hd:])
    xe_trans = jnp.concatenate([xt_lo, xt_hi], axis=0)

    out = _make_mlp()(Xe, xe_trans,
                      bn_g.reshape(1, D_MODEL), bn_b.reshape(1, D_MODEL),
                      W1, b1.reshape(1, D_MODEL), g1.reshape(1, D_MODEL),
                      bt1.reshape(1, D_MODEL),
                      W2, b2.reshape(1, D_MODEL), g2.reshape(1, D_MODEL),
                      bt2.reshape(1, D_MODEL))
    return out
